# baseline (speedup 1.0000x reference)
import functools

import jax
import jax.numpy as jnp
from jax import lax
from jax.experimental import pallas as pl
from jax.experimental.pallas import tpu as pltpu
from jax.experimental.pallas import tpu_sc as plsc

EMB = 64
NW = 32
CAND_CHUNKS = ((0, 104), (104, 104), (208, 92))


def _sc_pool(table, ctx_idx, cand_idx, B, C, LCTX, LC):
    SPW = B // NW
    NCI = C * LC
    PC = 2 * C
    PCP = PC + 1
    mesh = plsc.VectorSubcoreMesh(core_axis_name="c", subcore_axis_name="s")

    @functools.partial(
        pl.kernel,
        mesh=mesh,
        compiler_params=pltpu.CompilerParams(use_tc_tiling_on_sc=False,
                                             needs_layout_passes=False),
        out_type=[
            jax.ShapeDtypeStruct((EMB, B * C), jnp.float32),
            jax.ShapeDtypeStruct((B, EMB), jnp.float32),
        ],
        scratch_types=[
            pltpu.VMEM((SPW, LCTX), jnp.int32),
            pltpu.VMEM((SPW, NCI), jnp.int32),
            pltpu.VMEM((2, LCTX, EMB), jnp.float32),
            pltpu.VMEM((2, NCI, EMB), jnp.float32),
            pltpu.VMEM((2, EMB, PCP), jnp.float32),
            pltpu.VMEM((SPW, EMB), jnp.float32),
            pltpu.SemaphoreType.DMA,
            pltpu.SemaphoreType.DMA,
            pltpu.SemaphoreType.DMA,
            pltpu.SemaphoreType.DMA,
        ],
    )
    def sc_kernel(table_hbm, ctx_idx_hbm, cand_idx_hbm,
                  comb_hbm, ctxe_hbm,
                  ctx_idx_v, cand_idx_v, ctx_rows_v, cand_rows_v,
                  comb_v, ctxe_v, gsem0, gsem1, wsem0, wsem1):
        wid = lax.axis_index("s") * 2 + lax.axis_index("c")
        s0 = wid * SPW
        gsems = (gsem0, gsem1)
        wsems = (wsem0, wsem1)

        pltpu.sync_copy(ctx_idx_hbm.at[pl.ds(s0, SPW)], ctx_idx_v)
        pltpu.sync_copy(cand_idx_hbm.at[pl.ds(s0, SPW)], cand_idx_v)

        inv_ctx = jnp.float32(1.0 / LCTX)
        inv_lc = jnp.float32(1.0 / LC)
        zero = jnp.zeros((16,), jnp.float32)
        iota = lax.iota(jnp.int32, 16)
        riota = tuple(iota + 16 * ch for ch in range(4))

        def gather_descs(s, gb):
            descs = [pltpu.make_async_copy(
                table_hbm.at[ctx_idx_v.at[s]], ctx_rows_v.at[gb], gsems[gb])]
            for (off, ln) in CAND_CHUNKS:
                descs.append(pltpu.make_async_copy(
                    table_hbm.at[cand_idx_v.at[s, pl.ds(off, ln)]],
                    cand_rows_v.at[gb, pl.ds(off, ln)], gsems[gb]))
            return descs

        def issue(s, gb):
            for d in gather_descs(s, gb):
                d.start()

        def drain(s, gb):
            for d in gather_descs(s, gb):
                d.wait()

        def wdesc(p, wb):
            return pltpu.make_async_copy(
                comb_v.at[wb, pl.ds(0, EMB), pl.ds(0, PC)],
                comb_hbm.at[pl.ds(0, EMB), pl.ds((s0 + 2 * p) * C, PC)],
                wsems[wb])

        issue(0, 0)
        issue(1, 1)

        def pool_sample(s, gb, wb, u):
            drain(s, gb)

            rows_c = ctx_rows_v.at[gb]
            rows_k = cand_rows_v.at[gb]
            out_c = comb_v.at[wb]

            def ctx_red(i, accs):
                return tuple(
                    accs[ch] + rows_c[i, pl.ds(ch * 16, 16)]
                    for ch in range(4))
            sums = lax.fori_loop(0, LCTX, ctx_red, (zero, zero, zero, zero))
            ctx_m = tuple(a * inv_ctx for a in sums)
            for ch in range(4):
                ctxe_v[s, pl.ds(ch * 16, 16)] = ctx_m[ch]

            def cand_body(t, _):
                for v in range(4):
                    cidx = t * 4 + v
                    r = cidx * LC
                    col = jnp.full((16,), u * C + cidx, jnp.int32)
                    for ch in range(4):
                        acc = rows_k[r, pl.ds(ch * 16, 16)]
                        for k in range(1, LC):
                            acc = acc + rows_k[r + k, pl.ds(ch * 16, 16)]
                        plsc.store_scatter(out_c, [riota[ch], col],
                                           ctx_m[ch] + acc * inv_lc)
                return 0
            lax.fori_loop(0, C // 4, cand_body, 0)

            @pl.when(s + 2 < SPW)
            def _():
                issue(s + 2, gb)

        def loop_body(tt, carry):
            for wb in range(2):
                p = tt * 2 + wb
                s = p * 2

                @pl.when(p >= 2)
                def _():
                    wdesc(p - 2, wb).wait()

                pool_sample(s, 0, wb, 0)
                pool_sample(s + 1, 1, wb, 1)
                wdesc(p, wb).start()
            return carry

        lax.fori_loop(0, SPW // 4, loop_body, 0)

        wdesc(SPW // 2 - 2, 0).wait()
        wdesc(SPW // 2 - 1, 1).wait()

        pltpu.sync_copy(ctxe_v, ctxe_hbm.at[pl.ds(s0, SPW)])

    return sc_kernel(table, ctx_idx, cand_idx)


def _tc_term_head(x, W, b, W2, b2):
    N = x.shape[0]
    H = W.shape[1]
    OD = W2.shape[1]

    def body(x_ref, w_ref, b_ref, w2_ref, b2_ref, o_ref):
        h = jnp.dot(x_ref[...], w_ref[...],
                    preferred_element_type=jnp.float32) + b_ref[...]
        h = jnp.maximum(h, 0.0)
        o_ref[...] = jnp.dot(h, w2_ref[...],
                             preferred_element_type=jnp.float32) + b2_ref[...]

    return pl.pallas_call(
        body,
        grid=(1,),
        in_specs=[
            pl.BlockSpec((N, EMB), lambda i: (0, 0)),
            pl.BlockSpec((EMB, H), lambda i: (0, 0)),
            pl.BlockSpec((1, H), lambda i: (0, 0)),
            pl.BlockSpec((H, OD), lambda i: (0, 0)),
            pl.BlockSpec((1, OD), lambda i: (0, 0)),
        ],
        out_specs=pl.BlockSpec((N, OD), lambda i: (0, 0)),
        out_shape=jax.ShapeDtypeStruct((N, OD), jnp.float32),
    )(x, W, b, W2, b2)


def _tc_cand_head(xT, Wc1T, bc1c, Wc2T, bc2, block_cols):
    N = xT.shape[1]
    H = Wc1T.shape[0]
    grid = N // block_cols

    def body(x_ref, w1_ref, b1_ref, w2_ref, b2_ref, o_ref):
        h = jnp.dot(w1_ref[...], x_ref[...],
                    preferred_element_type=jnp.float32) + b1_ref[...]
        h = jnp.maximum(h, 0.0)
        s = jnp.dot(w2_ref[...], h,
                    preferred_element_type=jnp.float32) + b2_ref[...]
        o_ref[...] = s[None]

    return pl.pallas_call(
        body,
        grid=(grid,),
        in_specs=[
            pl.BlockSpec((EMB, block_cols), lambda i: (0, i)),
            pl.BlockSpec((H, EMB), lambda i: (0, 0)),
            pl.BlockSpec((H, 1), lambda i: (0, 0)),
            pl.BlockSpec((1, H), lambda i: (0, 0)),
            pl.BlockSpec((1, 1), lambda i: (0, 0)),
        ],
        out_specs=pl.BlockSpec((1, 1, block_cols), lambda i: (i, 0, 0)),
        out_shape=jax.ShapeDtypeStruct((grid, 1, block_cols), jnp.float32),
    )(xT, Wc1T, bc1c, Wc2T, bc2)


def kernel(table, W1, b1, W2, b2, Wc1, bc1, Wc2, bc2,
           context_indices, candidate_indices):
    B, LCTX = context_indices.shape
    _, C, LC = candidate_indices.shape

    ctx_i = context_indices.astype(jnp.int32)
    cand_i = candidate_indices.astype(jnp.int32).reshape(B, C * LC)

    combT, ctx_emb = _sc_pool(table, ctx_i, cand_i, B, C, LCTX, LC)

    term_logits = _tc_term_head(ctx_emb, W1, b1.reshape(1, -1),
                                W2, b2.reshape(1, -1))
    scores = _tc_cand_head(combT, Wc1.T, bc1.reshape(-1, 1),
                           Wc2.reshape(1, -1), bc2.reshape(1, 1), 2048)
    return term_logits, scores.reshape(B, C)


# speedup vs baseline: 1.2794x; 1.2794x over previous
import functools

import jax
import jax.numpy as jnp
from jax import lax
from jax.experimental import pallas as pl
from jax.experimental.pallas import tpu as pltpu
from jax.experimental.pallas import tpu_sc as plsc

EMB = 64
NW = 32
CAND_CHUNKS = ((0, 104), (104, 104), (208, 92))


def _sc_pool(table, ctx_idx, cand_idx, B, C, LCTX, LC):
    SPW = B // NW
    NCI = C * LC
    PC = 2 * C
    PCP = PC
    mesh = plsc.VectorSubcoreMesh(core_axis_name="c", subcore_axis_name="s")

    @functools.partial(
        pl.kernel,
        mesh=mesh,
        compiler_params=pltpu.CompilerParams(use_tc_tiling_on_sc=False,
                                             needs_layout_passes=False),
        out_type=[
            jax.ShapeDtypeStruct((EMB, B * C), jnp.float32),
            jax.ShapeDtypeStruct((B, EMB), jnp.float32),
        ],
        scratch_types=[
            pltpu.VMEM((SPW, LCTX), jnp.int32),
            pltpu.VMEM((SPW, NCI), jnp.int32),
            pltpu.VMEM((2, LCTX, EMB), jnp.float32),
            pltpu.VMEM((2, NCI, EMB), jnp.float32),
            pltpu.VMEM((2, EMB, PCP), jnp.float32),
            pltpu.VMEM((SPW, EMB), jnp.float32),
            pltpu.SemaphoreType.DMA,
            pltpu.SemaphoreType.DMA,
            pltpu.SemaphoreType.DMA,
            pltpu.SemaphoreType.DMA,
        ],
    )
    def sc_kernel(table_hbm, ctx_idx_hbm, cand_idx_hbm,
                  comb_hbm, ctxe_hbm,
                  ctx_idx_v, cand_idx_v, ctx_rows_v, cand_rows_v,
                  comb_v, ctxe_v, gsem0, gsem1, wsem0, wsem1):
        wid = lax.axis_index("s") * 2 + lax.axis_index("c")
        s0 = wid * SPW
        gsems = (gsem0, gsem1)
        wsems = (wsem0, wsem1)

        pltpu.sync_copy(ctx_idx_hbm.at[pl.ds(s0, SPW)], ctx_idx_v)
        pltpu.sync_copy(cand_idx_hbm.at[pl.ds(s0, SPW)], cand_idx_v)

        inv_ctx = jnp.float32(1.0 / LCTX)
        inv_lc = jnp.float32(1.0 / LC)
        zero = jnp.zeros((16,), jnp.float32)
        iota = lax.iota(jnp.int32, 16)
        riota = tuple(iota + 16 * ch for ch in range(4))

        def gather_descs(s, gb):
            descs = [pltpu.make_async_copy(
                table_hbm.at[ctx_idx_v.at[s]], ctx_rows_v.at[gb], gsems[gb])]
            for (off, ln) in CAND_CHUNKS:
                descs.append(pltpu.make_async_copy(
                    table_hbm.at[cand_idx_v.at[s, pl.ds(off, ln)]],
                    cand_rows_v.at[gb, pl.ds(off, ln)], gsems[gb]))
            return descs

        def issue(s, gb):
            for d in gather_descs(s, gb):
                d.start()

        def drain(s, gb):
            for d in gather_descs(s, gb):
                d.wait()

        def wdesc(p, wb):
            return pltpu.make_async_copy(
                comb_v.at[wb, pl.ds(0, EMB), pl.ds(0, PC)],
                comb_hbm.at[pl.ds(0, EMB), pl.ds((s0 + 2 * p) * C, PC)],
                wsems[wb])

        issue(0, 0)
        issue(1, 1)

        def pool_sample(s, gb, wb, u):
            drain(s, gb)

            rows_c = ctx_rows_v.at[gb]
            rows_k = cand_rows_v.at[gb]
            out_c = comb_v.at[wb]

            def ctx_red(i, accs):
                return tuple(
                    accs[ch] + rows_c[i, pl.ds(ch * 16, 16)]
                    for ch in range(4))
            sums = lax.fori_loop(0, LCTX, ctx_red, (zero, zero, zero, zero))
            ctx_m = tuple(a * inv_ctx for a in sums)
            for ch in range(4):
                ctxe_v[s, pl.ds(ch * 16, 16)] = ctx_m[ch]

            @plsc.parallel_loop(0, C, 1, unroll=4)
            def _(cidx):
                r = cidx * LC
                col = jnp.full((16,), u * C + cidx, jnp.int32)
                for ch in range(4):
                    acc = rows_k[r, pl.ds(ch * 16, 16)]
                    for k in range(1, LC):
                        acc = acc + rows_k[r + k, pl.ds(ch * 16, 16)]
                    plsc.store_scatter(out_c, [riota[ch], col],
                                       ctx_m[ch] + acc * inv_lc)

            @pl.when(s + 2 < SPW)
            def _():
                issue(s + 2, gb)

        def loop_body(tt, carry):
            for wb in range(2):
                p = tt * 2 + wb
                s = p * 2

                @pl.when(p >= 2)
                def _():
                    wdesc(p - 2, wb).wait()

                pool_sample(s, 0, wb, 0)
                pool_sample(s + 1, 1, wb, 1)
                wdesc(p, wb).start()
            return carry

        lax.fori_loop(0, SPW // 4, loop_body, 0)

        wdesc(SPW // 2 - 2, 0).wait()
        wdesc(SPW // 2 - 1, 1).wait()

        pltpu.sync_copy(ctxe_v, ctxe_hbm.at[pl.ds(s0, SPW)])

    return sc_kernel(table, ctx_idx, cand_idx)


def _tc_term_head(x, W, b, W2, b2):
    N = x.shape[0]
    H = W.shape[1]
    OD = W2.shape[1]

    def body(x_ref, w_ref, b_ref, w2_ref, b2_ref, o_ref):
        h = jnp.dot(x_ref[...], w_ref[...],
                    preferred_element_type=jnp.float32) + b_ref[...]
        h = jnp.maximum(h, 0.0)
        o_ref[...] = jnp.dot(h, w2_ref[...],
                             preferred_element_type=jnp.float32) + b2_ref[...]

    return pl.pallas_call(
        body,
        grid=(1,),
        in_specs=[
            pl.BlockSpec((N, EMB), lambda i: (0, 0)),
            pl.BlockSpec((EMB, H), lambda i: (0, 0)),
            pl.BlockSpec((1, H), lambda i: (0, 0)),
            pl.BlockSpec((H, OD), lambda i: (0, 0)),
            pl.BlockSpec((1, OD), lambda i: (0, 0)),
        ],
        out_specs=pl.BlockSpec((N, OD), lambda i: (0, 0)),
        out_shape=jax.ShapeDtypeStruct((N, OD), jnp.float32),
    )(x, W, b, W2, b2)


def _tc_cand_head(xT, Wc1T, bc1c, Wc2T, bc2, block_cols):
    N = xT.shape[1]
    H = Wc1T.shape[0]
    grid = N // block_cols

    def body(x_ref, w1_ref, b1_ref, w2_ref, b2_ref, o_ref):
        h = jnp.dot(w1_ref[...], x_ref[...],
                    preferred_element_type=jnp.float32) + b1_ref[...]
        h = jnp.maximum(h, 0.0)
        s = jnp.dot(w2_ref[...], h,
                    preferred_element_type=jnp.float32) + b2_ref[...]
        o_ref[...] = s[None]

    return pl.pallas_call(
        body,
        grid=(grid,),
        in_specs=[
            pl.BlockSpec((EMB, block_cols), lambda i: (0, i)),
            pl.BlockSpec((H, EMB), lambda i: (0, 0)),
            pl.BlockSpec((H, 1), lambda i: (0, 0)),
            pl.BlockSpec((1, H), lambda i: (0, 0)),
            pl.BlockSpec((1, 1), lambda i: (0, 0)),
        ],
        out_specs=pl.BlockSpec((1, 1, block_cols), lambda i: (i, 0, 0)),
        out_shape=jax.ShapeDtypeStruct((grid, 1, block_cols), jnp.float32),
    )(xT, Wc1T, bc1c, Wc2T, bc2)


def kernel(table, W1, b1, W2, b2, Wc1, bc1, Wc2, bc2,
           context_indices, candidate_indices):
    B, LCTX = context_indices.shape
    _, C, LC = candidate_indices.shape

    ctx_i = context_indices.astype(jnp.int32)
    cand_i = candidate_indices.astype(jnp.int32).reshape(B, C * LC)

    combT, ctx_emb = _sc_pool(table, ctx_i, cand_i, B, C, LCTX, LC)

    term_logits = _tc_term_head(ctx_emb, W1, b1.reshape(1, -1),
                                W2, b2.reshape(1, -1))
    scores = _tc_cand_head(combT, Wc1.T, bc1.reshape(-1, 1),
                           Wc2.reshape(1, -1), bc2.reshape(1, 1), 2048)
    return term_logits, scores.reshape(B, C)


# speedup vs baseline: 1.2831x; 1.0029x over previous
import functools

import jax
import jax.numpy as jnp
from jax import lax
from jax.experimental import pallas as pl
from jax.experimental.pallas import tpu as pltpu
from jax.experimental.pallas import tpu_sc as plsc

EMB = 64
NW = 32
CAND_CHUNKS = ((0, 104), (104, 104), (208, 92))


def _sc_pool(table, ctx_idx, cand_idx, B, C, LCTX, LC):
    SPW = B // NW
    NCI = C * LC
    PC = 2 * C
    PCP = PC
    mesh = plsc.VectorSubcoreMesh(core_axis_name="c", subcore_axis_name="s")

    @functools.partial(
        pl.kernel,
        mesh=mesh,
        compiler_params=pltpu.CompilerParams(use_tc_tiling_on_sc=False,
                                             needs_layout_passes=False),
        out_type=[
            jax.ShapeDtypeStruct((EMB, B * C), jnp.float32),
            jax.ShapeDtypeStruct((B, EMB), jnp.float32),
        ],
        scratch_types=[
            pltpu.VMEM((SPW, LCTX), jnp.int32),
            pltpu.VMEM((SPW, NCI), jnp.int32),
            pltpu.VMEM((2, LCTX, EMB), jnp.float32),
            pltpu.VMEM((2, NCI, EMB), jnp.float32),
            pltpu.VMEM((2, EMB, PCP), jnp.float32),
            pltpu.VMEM((SPW, EMB), jnp.float32),
            pltpu.SemaphoreType.DMA,
            pltpu.SemaphoreType.DMA,
            pltpu.SemaphoreType.DMA,
            pltpu.SemaphoreType.DMA,
        ],
    )
    def sc_kernel(table_hbm, ctx_idx_hbm, cand_idx_hbm,
                  comb_hbm, ctxe_hbm,
                  ctx_idx_v, cand_idx_v, ctx_rows_v, cand_rows_v,
                  comb_v, ctxe_v, gsem0, gsem1, wsem0, wsem1):
        wid = lax.axis_index("s") * 2 + lax.axis_index("c")
        s0 = wid * SPW
        gsems = (gsem0, gsem1)
        wsems = (wsem0, wsem1)

        pltpu.sync_copy(ctx_idx_hbm.at[pl.ds(s0, SPW)], ctx_idx_v)
        pltpu.sync_copy(cand_idx_hbm.at[pl.ds(s0, SPW)], cand_idx_v)

        inv_ctx = jnp.float32(1.0 / LCTX)
        inv_lc = jnp.float32(1.0 / LC)
        zero = jnp.zeros((16,), jnp.float32)
        iota = lax.iota(jnp.int32, 16)
        riota = tuple(iota + 16 * ch for ch in range(4))

        def gather_descs(s, gb):
            descs = [pltpu.make_async_copy(
                table_hbm.at[ctx_idx_v.at[s]], ctx_rows_v.at[gb], gsems[gb])]
            for (off, ln) in CAND_CHUNKS:
                descs.append(pltpu.make_async_copy(
                    table_hbm.at[cand_idx_v.at[s, pl.ds(off, ln)]],
                    cand_rows_v.at[gb, pl.ds(off, ln)], gsems[gb]))
            return descs

        def issue(s, gb):
            for d in gather_descs(s, gb):
                d.start()

        def drain(s, gb):
            for d in gather_descs(s, gb):
                d.wait()

        def wdesc(p, wb):
            return pltpu.make_async_copy(
                comb_v.at[wb, pl.ds(0, EMB), pl.ds(0, PC)],
                comb_hbm.at[pl.ds(0, EMB), pl.ds((s0 + 2 * p) * C, PC)],
                wsems[wb])

        issue(0, 0)
        issue(1, 1)

        def pool_sample(s, gb, wb, u):
            drain(s, gb)

            rows_c = ctx_rows_v.at[gb]
            rows_k = cand_rows_v.at[gb]
            out_c = comb_v.at[wb]

            @plsc.parallel_loop(0, LCTX, 2, unroll=2,
                                carry=(zero,) * 8)
            def sums(i, accs):
                a = tuple(
                    accs[ch] + rows_c[i, pl.ds(ch * 16, 16)]
                    for ch in range(4))
                b = tuple(
                    accs[4 + ch] + rows_c[i + 1, pl.ds(ch * 16, 16)]
                    for ch in range(4))
                return a + b
            ctx_m = tuple(
                (sums[ch] + sums[4 + ch]) * inv_ctx for ch in range(4))
            for ch in range(4):
                ctxe_v[s, pl.ds(ch * 16, 16)] = ctx_m[ch]

            @plsc.parallel_loop(0, C, 1, unroll=4)
            def _(cidx):
                r = cidx * LC
                col = jnp.full((16,), u * C + cidx, jnp.int32)
                for ch in range(4):
                    acc = rows_k[r, pl.ds(ch * 16, 16)]
                    for k in range(1, LC):
                        acc = acc + rows_k[r + k, pl.ds(ch * 16, 16)]
                    plsc.store_scatter(out_c, [riota[ch], col],
                                       ctx_m[ch] + acc * inv_lc)

            @pl.when(s + 2 < SPW)
            def _():
                issue(s + 2, gb)

        def loop_body(tt, carry):
            for wb in range(2):
                p = tt * 2 + wb
                s = p * 2

                @pl.when(p >= 2)
                def _():
                    wdesc(p - 2, wb).wait()

                pool_sample(s, 0, wb, 0)
                pool_sample(s + 1, 1, wb, 1)
                wdesc(p, wb).start()
            return carry

        lax.fori_loop(0, SPW // 4, loop_body, 0)

        wdesc(SPW // 2 - 2, 0).wait()
        wdesc(SPW // 2 - 1, 1).wait()

        pltpu.sync_copy(ctxe_v, ctxe_hbm.at[pl.ds(s0, SPW)])

    return sc_kernel(table, ctx_idx, cand_idx)


def _tc_term_head(x, W, b, W2, b2):
    N = x.shape[0]
    H = W.shape[1]
    OD = W2.shape[1]

    def body(x_ref, w_ref, b_ref, w2_ref, b2_ref, o_ref):
        h = jnp.dot(x_ref[...], w_ref[...],
                    preferred_element_type=jnp.float32) + b_ref[...]
        h = jnp.maximum(h, 0.0)
        o_ref[...] = jnp.dot(h, w2_ref[...],
                             preferred_element_type=jnp.float32) + b2_ref[...]

    return pl.pallas_call(
        body,
        grid=(1,),
        in_specs=[
            pl.BlockSpec((N, EMB), lambda i: (0, 0)),
            pl.BlockSpec((EMB, H), lambda i: (0, 0)),
            pl.BlockSpec((1, H), lambda i: (0, 0)),
            pl.BlockSpec((H, OD), lambda i: (0, 0)),
            pl.BlockSpec((1, OD), lambda i: (0, 0)),
        ],
        out_specs=pl.BlockSpec((N, OD), lambda i: (0, 0)),
        out_shape=jax.ShapeDtypeStruct((N, OD), jnp.float32),
    )(x, W, b, W2, b2)


def _tc_cand_head(xT, Wc1T, bc1c, Wc2T, bc2, block_cols):
    N = xT.shape[1]
    H = Wc1T.shape[0]
    grid = N // block_cols

    def body(x_ref, w1_ref, b1_ref, w2_ref, b2_ref, o_ref):
        h = jnp.dot(w1_ref[...], x_ref[...],
                    preferred_element_type=jnp.float32) + b1_ref[...]
        h = jnp.maximum(h, 0.0)
        s = jnp.dot(w2_ref[...], h,
                    preferred_element_type=jnp.float32) + b2_ref[...]
        o_ref[...] = s[None]

    return pl.pallas_call(
        body,
        grid=(grid,),
        in_specs=[
            pl.BlockSpec((EMB, block_cols), lambda i: (0, i)),
            pl.BlockSpec((H, EMB), lambda i: (0, 0)),
            pl.BlockSpec((H, 1), lambda i: (0, 0)),
            pl.BlockSpec((1, H), lambda i: (0, 0)),
            pl.BlockSpec((1, 1), lambda i: (0, 0)),
        ],
        out_specs=pl.BlockSpec((1, 1, block_cols), lambda i: (i, 0, 0)),
        out_shape=jax.ShapeDtypeStruct((grid, 1, block_cols), jnp.float32),
    )(xT, Wc1T, bc1c, Wc2T, bc2)


def kernel(table, W1, b1, W2, b2, Wc1, bc1, Wc2, bc2,
           context_indices, candidate_indices):
    B, LCTX = context_indices.shape
    _, C, LC = candidate_indices.shape

    ctx_i = context_indices.astype(jnp.int32)
    cand_i = candidate_indices.astype(jnp.int32).reshape(B, C * LC)

    combT, ctx_emb = _sc_pool(table, ctx_i, cand_i, B, C, LCTX, LC)

    term_logits = _tc_term_head(ctx_emb, W1, b1.reshape(1, -1),
                                W2, b2.reshape(1, -1))
    scores = _tc_cand_head(combT, Wc1.T, bc1.reshape(-1, 1),
                           Wc2.reshape(1, -1), bc2.reshape(1, 1), 2048)
    return term_logits, scores.reshape(B, C)


# speedup vs baseline: 1.3970x; 1.0887x over previous
import functools

import jax
import jax.numpy as jnp
from jax import lax
from jax.experimental import pallas as pl
from jax.experimental.pallas import tpu as pltpu
from jax.experimental.pallas import tpu_sc as plsc

EMB = 64
NW = 32
CAND_CHUNKS = ((0, 104), (104, 104), (208, 92))


def _sc_pool(table, ctx_idx, cand_idx, B, C, LCTX, LC):
    SPW = B // NW
    NCI = C * LC
    PC = 2 * C
    PCP = PC
    mesh = plsc.VectorSubcoreMesh(core_axis_name="c", subcore_axis_name="s")

    @functools.partial(
        pl.kernel,
        mesh=mesh,
        compiler_params=pltpu.CompilerParams(use_tc_tiling_on_sc=False,
                                             needs_layout_passes=False),
        out_type=[
            jax.ShapeDtypeStruct((EMB, B * C), jnp.float32),
            jax.ShapeDtypeStruct((B, EMB), jnp.float32),
        ],
        scratch_types=[
            pltpu.VMEM((SPW, LCTX), jnp.int32),
            pltpu.VMEM((SPW, NCI), jnp.int32),
            pltpu.VMEM((2, LCTX, EMB), jnp.float32),
            pltpu.VMEM((2, NCI, EMB), jnp.float32),
            pltpu.VMEM((2, EMB, PCP), jnp.float32),
            pltpu.VMEM((SPW, EMB), jnp.float32),
            pltpu.SemaphoreType.DMA,
            pltpu.SemaphoreType.DMA,
            pltpu.SemaphoreType.DMA,
            pltpu.SemaphoreType.DMA,
        ],
    )
    def sc_kernel(table_hbm, ctx_idx_hbm, cand_idx_hbm,
                  comb_hbm, ctxe_hbm,
                  ctx_idx_v, cand_idx_v, ctx_rows_v, cand_rows_v,
                  comb_v, ctxe_v, gsem0, gsem1, wsem0, wsem1):
        wid = lax.axis_index("s") * 2 + lax.axis_index("c")
        s0 = wid * SPW
        gsems = (gsem0, gsem1)
        wsems = (wsem0, wsem1)

        pltpu.sync_copy(ctx_idx_hbm.at[pl.ds(s0, SPW)], ctx_idx_v)
        pltpu.sync_copy(cand_idx_hbm.at[pl.ds(s0, SPW)], cand_idx_v)

        inv_ctx = jnp.float32(1.0 / LCTX)
        inv_lc = jnp.float32(1.0 / LC)
        zero = jnp.zeros((16,), jnp.float32)
        iota = lax.iota(jnp.int32, 16)
        riota = tuple(iota + 16 * ch for ch in range(4))

        def gather_descs(s, gb):
            descs = [pltpu.make_async_copy(
                table_hbm.at[ctx_idx_v.at[s]], ctx_rows_v.at[gb], gsems[gb])]
            for (off, ln) in CAND_CHUNKS:
                descs.append(pltpu.make_async_copy(
                    table_hbm.at[cand_idx_v.at[s, pl.ds(off, ln)]],
                    cand_rows_v.at[gb, pl.ds(off, ln)], gsems[gb]))
            return descs

        def issue(s, gb):
            for d in gather_descs(s, gb):
                d.start()

        def drain(s, gb):
            for d in gather_descs(s, gb):
                d.wait()

        def wdesc(p, wb):
            return pltpu.make_async_copy(
                comb_v.at[wb, pl.ds(0, EMB), pl.ds(0, PC)],
                comb_hbm.at[pl.ds(0, EMB), pl.ds((s0 + 2 * p) * C, PC)],
                wsems[wb])

        issue(0, 0)
        issue(1, 1)

        def pool_sample(s, gb, wb, u):
            drain(s, gb)

            rows_c = ctx_rows_v.at[gb]
            rows_k = cand_rows_v.at[gb]
            out_c = comb_v.at[wb]

            @plsc.parallel_loop(0, LCTX, 2, unroll=2,
                                carry=(zero,) * 8)
            def sums(i, accs):
                a = tuple(
                    accs[ch] + rows_c[i, pl.ds(ch * 16, 16)]
                    for ch in range(4))
                b = tuple(
                    accs[4 + ch] + rows_c[i + 1, pl.ds(ch * 16, 16)]
                    for ch in range(4))
                return a + b
            ctx_m = tuple(
                (sums[ch] + sums[4 + ch]) * inv_ctx for ch in range(4))
            for ch in range(4):
                ctxe_v[s, pl.ds(ch * 16, 16)] = ctx_m[ch]

            @plsc.parallel_loop(0, C, 1, unroll=4)
            def _(cidx):
                r = cidx * LC
                col = jnp.full((16,), u * C + cidx, jnp.int32)
                for ch in range(4):
                    acc = rows_k[r, pl.ds(ch * 16, 16)]
                    for k in range(1, LC):
                        acc = acc + rows_k[r + k, pl.ds(ch * 16, 16)]
                    plsc.store_scatter(out_c, [riota[ch], col],
                                       ctx_m[ch] + acc * inv_lc)

            @pl.when(s + 2 < SPW)
            def _():
                issue(s + 2, gb)

        def loop_body(tt, carry):
            for wb in range(2):
                p = tt * 2 + wb
                s = p * 2

                @pl.when(p >= 2)
                def _():
                    wdesc(p - 2, wb).wait()

                pool_sample(s, 0, wb, 0)
                pool_sample(s + 1, 1, wb, 1)
                wdesc(p, wb).start()
            return carry

        lax.fori_loop(0, SPW // 4, loop_body, 0)

        wdesc(SPW // 2 - 2, 0).wait()
        wdesc(SPW // 2 - 1, 1).wait()

        pltpu.sync_copy(ctxe_v, ctxe_hbm.at[pl.ds(s0, SPW)])

    return sc_kernel(table, ctx_idx, cand_idx)


def _tc_term_head(x, W, b, W2, b2):
    N = x.shape[0]
    H = W.shape[1]
    OD = W2.shape[1]

    def body(x_ref, w_ref, b_ref, w2_ref, b2_ref, o_ref):
        h = jnp.dot(x_ref[...], w_ref[...],
                    preferred_element_type=jnp.float32) + b_ref[...]
        h = jnp.maximum(h, 0.0)
        o_ref[...] = jnp.dot(h, w2_ref[...],
                             preferred_element_type=jnp.float32) + b2_ref[...]

    return pl.pallas_call(
        body,
        grid=(1,),
        in_specs=[
            pl.BlockSpec((N, EMB), lambda i: (0, 0)),
            pl.BlockSpec((EMB, H), lambda i: (0, 0)),
            pl.BlockSpec((1, H), lambda i: (0, 0)),
            pl.BlockSpec((H, OD), lambda i: (0, 0)),
            pl.BlockSpec((1, OD), lambda i: (0, 0)),
        ],
        out_specs=pl.BlockSpec((N, OD), lambda i: (0, 0)),
        out_shape=jax.ShapeDtypeStruct((N, OD), jnp.float32),
    )(x, W, b, W2, b2)


def _tc_cand_head(xT3, Wc1T, bc1c, Wc2T, bc2, block_groups):
    G = xT3.shape[1]
    H = Wc1T.shape[0]
    grid = G // block_groups

    def body(x_ref, w1_ref, b1_ref, w2_ref, b2_ref, o_ref):
        for g in range(block_groups):
            h = jnp.dot(w1_ref[...], x_ref[:, g, :],
                        preferred_element_type=jnp.float32) + b1_ref[...]
            h = jnp.maximum(h, 0.0)
            s = jnp.dot(w2_ref[...], h,
                        preferred_element_type=jnp.float32) + b2_ref[...]
            o_ref[0, g, :] = s[0]

    return pl.pallas_call(
        body,
        grid=(grid,),
        in_specs=[
            pl.BlockSpec((EMB, block_groups, 128), lambda i: (0, i, 0)),
            pl.BlockSpec((H, EMB), lambda i: (0, 0)),
            pl.BlockSpec((H, 1), lambda i: (0, 0)),
            pl.BlockSpec((1, H), lambda i: (0, 0)),
            pl.BlockSpec((1, 1), lambda i: (0, 0)),
        ],
        out_specs=pl.BlockSpec((1, block_groups, 128), lambda i: (i, 0, 0)),
        out_shape=jax.ShapeDtypeStruct((grid, block_groups, 128),
                                       jnp.float32),
    )(xT3, Wc1T, bc1c, Wc2T, bc2)


def kernel(table, W1, b1, W2, b2, Wc1, bc1, Wc2, bc2,
           context_indices, candidate_indices):
    B, LCTX = context_indices.shape
    _, C, LC = candidate_indices.shape

    ctx_i = context_indices.astype(jnp.int32)
    cand_i = candidate_indices.astype(jnp.int32).reshape(B, C * LC)

    combT, ctx_emb = _sc_pool(table, ctx_i, cand_i, B, C, LCTX, LC)
    combT3 = combT.reshape(EMB, B * C // 128, 128)

    term_logits = _tc_term_head(ctx_emb, W1, b1.reshape(1, -1),
                                W2, b2.reshape(1, -1))
    scores = _tc_cand_head(combT3, Wc1.T, bc1.reshape(-1, 1),
                           Wc2.reshape(1, -1), bc2.reshape(1, 1), 16)
    return term_logits, scores.reshape(B, C)


# speedup vs baseline: 1.4576x; 1.0433x over previous
import functools

import jax
import jax.numpy as jnp
from jax import lax
from jax.experimental import pallas as pl
from jax.experimental.pallas import tpu as pltpu
from jax.experimental.pallas import tpu_sc as plsc

EMB = 64
NW = 32
CAND_CHUNKS = ((0, 104), (104, 104), (208, 92))


def _sc_pool(table, ctx_idx, cand_idx, B, C, LCTX, LC):
    SPW = B // NW
    NCI = C * LC
    PC = 2 * C
    PCP = PC
    mesh = plsc.VectorSubcoreMesh(core_axis_name="c", subcore_axis_name="s")

    @functools.partial(
        pl.kernel,
        mesh=mesh,
        compiler_params=pltpu.CompilerParams(use_tc_tiling_on_sc=False,
                                             needs_layout_passes=False),
        out_type=[
            jax.ShapeDtypeStruct((EMB, B * C), jnp.float32),
            jax.ShapeDtypeStruct((B, EMB), jnp.float32),
        ],
        scratch_types=[
            pltpu.VMEM((SPW, LCTX), jnp.int32),
            pltpu.VMEM((SPW, NCI), jnp.int32),
            pltpu.VMEM((2, LCTX, EMB), jnp.float32),
            pltpu.VMEM((2, NCI, EMB), jnp.float32),
            pltpu.VMEM((2, EMB, PCP), jnp.float32),
            pltpu.VMEM((SPW, EMB), jnp.float32),
            pltpu.SemaphoreType.DMA,
            pltpu.SemaphoreType.DMA,
            pltpu.SemaphoreType.DMA,
            pltpu.SemaphoreType.DMA,
        ],
    )
    def sc_kernel(table_hbm, ctx_idx_hbm, cand_idx_hbm,
                  comb_hbm, ctxe_hbm,
                  ctx_idx_v, cand_idx_v, ctx_rows_v, cand_rows_v,
                  comb_v, ctxe_v, gsem0, gsem1, wsem0, wsem1):
        wid = lax.axis_index("s") * 2 + lax.axis_index("c")
        s0 = wid * SPW
        gsems = (gsem0, gsem1)
        wsems = (wsem0, wsem1)

        pltpu.sync_copy(ctx_idx_hbm.at[pl.ds(s0, SPW)], ctx_idx_v)
        pltpu.sync_copy(cand_idx_hbm.at[pl.ds(s0, SPW)], cand_idx_v)

        inv_ctx = jnp.float32(1.0 / LCTX)
        inv_lc = jnp.float32(1.0 / LC)
        zero = jnp.zeros((16,), jnp.float32)
        iota = lax.iota(jnp.int32, 16)
        riota = tuple(iota + 16 * ch for ch in range(4))

        def gather_descs(s, gb):
            descs = [pltpu.make_async_copy(
                table_hbm.at[ctx_idx_v.at[s]], ctx_rows_v.at[gb], gsems[gb])]
            for (off, ln) in CAND_CHUNKS:
                descs.append(pltpu.make_async_copy(
                    table_hbm.at[cand_idx_v.at[s, pl.ds(off, ln)]],
                    cand_rows_v.at[gb, pl.ds(off, ln)], gsems[gb]))
            return descs

        def issue(s, gb):
            for d in gather_descs(s, gb):
                d.start()

        def drain(s, gb):
            for d in gather_descs(s, gb):
                d.wait()

        def wdesc(p, wb):
            return pltpu.make_async_copy(
                comb_v.at[wb, pl.ds(0, EMB), pl.ds(0, PC)],
                comb_hbm.at[pl.ds(0, EMB), pl.ds((s0 + 2 * p) * C, PC)],
                wsems[wb])

        issue(0, 0)
        issue(1, 1)

        def pool_sample(s, gb, wb, u):
            drain(s, gb)

            rows_c = ctx_rows_v.at[gb]
            rows_k = cand_rows_v.at[gb]
            out_c = comb_v.at[wb]

            @plsc.parallel_loop(0, LCTX, 2, unroll=2,
                                carry=(zero,) * 8)
            def sums(i, accs):
                a = tuple(
                    accs[ch] + rows_c[i, pl.ds(ch * 16, 16)]
                    for ch in range(4))
                b = tuple(
                    accs[4 + ch] + rows_c[i + 1, pl.ds(ch * 16, 16)]
                    for ch in range(4))
                return a + b
            ctx_m = tuple(
                (sums[ch] + sums[4 + ch]) * inv_ctx for ch in range(4))
            for ch in range(4):
                ctxe_v[s, pl.ds(ch * 16, 16)] = ctx_m[ch]

            @plsc.parallel_loop(0, C, 1, unroll=4)
            def _(cidx):
                r = cidx * LC
                col = jnp.full((16,), u * C + cidx, jnp.int32)
                for ch in range(4):
                    acc = rows_k[r, pl.ds(ch * 16, 16)]
                    for k in range(1, LC):
                        acc = acc + rows_k[r + k, pl.ds(ch * 16, 16)]
                    plsc.store_scatter(out_c, [riota[ch], col],
                                       ctx_m[ch] + acc * inv_lc)

            @pl.when(s + 2 < SPW)
            def _():
                issue(s + 2, gb)

        def loop_body(tt, carry):
            for wb in range(2):
                p = tt * 2 + wb
                s = p * 2

                @pl.when(p >= 2)
                def _():
                    wdesc(p - 2, wb).wait()

                pool_sample(s, 0, wb, 0)
                pool_sample(s + 1, 1, wb, 1)
                wdesc(p, wb).start()
            return carry

        lax.fori_loop(0, SPW // 4, loop_body, 0)

        wdesc(SPW // 2 - 2, 0).wait()
        wdesc(SPW // 2 - 1, 1).wait()

        pltpu.sync_copy(ctxe_v, ctxe_hbm.at[pl.ds(s0, SPW)])

    return sc_kernel(table, ctx_idx, cand_idx)


def _tc_term_head(x, W, b, W2, b2):
    N = x.shape[0]
    H = W.shape[1]
    OD = W2.shape[1]

    def body(x_ref, w_ref, b_ref, w2_ref, b2_ref, o_ref):
        h = jnp.dot(x_ref[...], w_ref[...],
                    preferred_element_type=jnp.float32) + b_ref[...]
        h = jnp.maximum(h, 0.0)
        o_ref[...] = jnp.dot(h, w2_ref[...],
                             preferred_element_type=jnp.float32) + b2_ref[...]

    return pl.pallas_call(
        body,
        grid=(1,),
        in_specs=[
            pl.BlockSpec((N, EMB), lambda i: (0, 0)),
            pl.BlockSpec((EMB, H), lambda i: (0, 0)),
            pl.BlockSpec((1, H), lambda i: (0, 0)),
            pl.BlockSpec((H, OD), lambda i: (0, 0)),
            pl.BlockSpec((1, OD), lambda i: (0, 0)),
        ],
        out_specs=pl.BlockSpec((N, OD), lambda i: (0, 0)),
        out_shape=jax.ShapeDtypeStruct((N, OD), jnp.float32),
    )(x, W, b, W2, b2)


def _tc_cand_head(xT3, Wc1T, bc1c, Wc2T, bc2, block_groups):
    G = xT3.shape[1]
    H = Wc1T.shape[0]
    grid = G // block_groups

    def body(x_ref, w1_ref, b1_ref, w2_ref, b2_ref, o_ref):
        for g in range(block_groups):
            h = jnp.dot(w1_ref[...], x_ref[:, g, :],
                        preferred_element_type=jnp.float32) + b1_ref[...]
            h = jnp.maximum(h, 0.0)
            s = jnp.dot(w2_ref[...], h,
                        preferred_element_type=jnp.float32) + b2_ref[...]
            o_ref[0, g, :] = s[0]

    return pl.pallas_call(
        body,
        grid=(grid,),
        in_specs=[
            pl.BlockSpec((EMB, block_groups, 128), lambda i: (0, i, 0)),
            pl.BlockSpec((H, EMB), lambda i: (0, 0)),
            pl.BlockSpec((H, 1), lambda i: (0, 0)),
            pl.BlockSpec((1, H), lambda i: (0, 0)),
            pl.BlockSpec((1, 1), lambda i: (0, 0)),
        ],
        out_specs=pl.BlockSpec((1, block_groups, 128), lambda i: (i, 0, 0)),
        out_shape=jax.ShapeDtypeStruct((grid, block_groups, 128),
                                       jnp.float32),
    )(xT3, Wc1T, bc1c, Wc2T, bc2)


def kernel(table, W1, b1, W2, b2, Wc1, bc1, Wc2, bc2,
           context_indices, candidate_indices):
    B, LCTX = context_indices.shape
    _, C, LC = candidate_indices.shape

    ctx_i = context_indices.astype(jnp.int32)
    cand_i = candidate_indices.astype(jnp.int32).reshape(B, C * LC)

    combT, ctx_emb = _sc_pool(table, ctx_i, cand_i, B, C, LCTX, LC)
    combT3 = combT.reshape(EMB, B * C // 128, 128)

    term_logits = _tc_term_head(ctx_emb, W1, b1.reshape(1, -1),
                                W2, b2.reshape(1, -1))
    scores = _tc_cand_head(combT3, Wc1.T, bc1.reshape(-1, 1),
                           Wc2.reshape(1, -1), bc2.reshape(1, 1), 32)
    return term_logits, scores.reshape(B, C)


# speedup vs baseline: 1.4780x; 1.0140x over previous
import functools

import jax
import jax.numpy as jnp
from jax import lax
from jax.experimental import pallas as pl
from jax.experimental.pallas import tpu as pltpu
from jax.experimental.pallas import tpu_sc as plsc

EMB = 64
NW = 32
CAND_CHUNKS = ((0, 104), (104, 104), (208, 92))


def _sc_pool(table, ctx_idx, cand_idx, B, C, LCTX, LC):
    SPW = B // NW
    NCI = C * LC
    PC = 2 * C
    PCP = PC
    mesh = plsc.VectorSubcoreMesh(core_axis_name="c", subcore_axis_name="s")

    @functools.partial(
        pl.kernel,
        mesh=mesh,
        compiler_params=pltpu.CompilerParams(use_tc_tiling_on_sc=False,
                                             needs_layout_passes=False),
        out_type=[
            jax.ShapeDtypeStruct((EMB, B * C), jnp.float32),
            jax.ShapeDtypeStruct((B, EMB), jnp.float32),
        ],
        scratch_types=[
            pltpu.VMEM((SPW, LCTX), jnp.int32),
            pltpu.VMEM((SPW, NCI), jnp.int32),
            pltpu.VMEM((2, LCTX, EMB), jnp.float32),
            pltpu.VMEM((2, NCI, EMB), jnp.float32),
            pltpu.VMEM((2, EMB, PCP), jnp.float32),
            pltpu.VMEM((SPW, EMB), jnp.float32),
            pltpu.SemaphoreType.DMA,
            pltpu.SemaphoreType.DMA,
            pltpu.SemaphoreType.DMA,
            pltpu.SemaphoreType.DMA,
        ],
    )
    def sc_kernel(table_hbm, ctx_idx_hbm, cand_idx_hbm,
                  comb_hbm, ctxe_hbm,
                  ctx_idx_v, cand_idx_v, ctx_rows_v, cand_rows_v,
                  comb_v, ctxe_v, gsem0, gsem1, wsem0, wsem1):
        wid = lax.axis_index("s") * 2 + lax.axis_index("c")
        s0 = wid * SPW
        gsems = (gsem0, gsem1)
        wsems = (wsem0, wsem1)

        pltpu.sync_copy(ctx_idx_hbm.at[pl.ds(s0, SPW)], ctx_idx_v)
        pltpu.sync_copy(cand_idx_hbm.at[pl.ds(s0, SPW)], cand_idx_v)

        inv_ctx = jnp.float32(1.0 / LCTX)
        inv_lc = jnp.float32(1.0 / LC)
        zero = jnp.zeros((16,), jnp.float32)
        iota = lax.iota(jnp.int32, 16)
        riota = tuple(iota + 16 * ch for ch in range(4))

        def gather_descs(s, gb):
            descs = [pltpu.make_async_copy(
                table_hbm.at[ctx_idx_v.at[s]], ctx_rows_v.at[gb], gsems[gb])]
            for (off, ln) in CAND_CHUNKS:
                descs.append(pltpu.make_async_copy(
                    table_hbm.at[cand_idx_v.at[s, pl.ds(off, ln)]],
                    cand_rows_v.at[gb, pl.ds(off, ln)], gsems[gb]))
            return descs

        def issue(s, gb):
            for d in gather_descs(s, gb):
                d.start()

        def drain(s, gb):
            for d in gather_descs(s, gb):
                d.wait()

        def wdesc(p, wb):
            return pltpu.make_async_copy(
                comb_v.at[wb, pl.ds(0, EMB), pl.ds(0, PC)],
                comb_hbm.at[pl.ds(0, EMB), pl.ds((s0 + 2 * p) * C, PC)],
                wsems[wb])

        issue(0, 0)
        issue(1, 1)

        def pool_sample(s, gb, wb, u):
            drain(s, gb)

            rows_c = ctx_rows_v.at[gb]
            rows_k = cand_rows_v.at[gb]
            out_c = comb_v.at[wb]

            @plsc.parallel_loop(0, LCTX, 2, unroll=2,
                                carry=(zero,) * 8)
            def sums(i, accs):
                a = tuple(
                    accs[ch] + rows_c[i, pl.ds(ch * 16, 16)]
                    for ch in range(4))
                b = tuple(
                    accs[4 + ch] + rows_c[i + 1, pl.ds(ch * 16, 16)]
                    for ch in range(4))
                return a + b
            ctx_m = tuple(
                (sums[ch] + sums[4 + ch]) * inv_ctx for ch in range(4))
            for ch in range(4):
                ctxe_v[s, pl.ds(ch * 16, 16)] = ctx_m[ch]

            @plsc.parallel_loop(0, C, 1, unroll=4)
            def _(cidx):
                r = cidx * LC
                col = jnp.full((16,), u * C + cidx, jnp.int32)
                for ch in range(4):
                    acc = rows_k[r, pl.ds(ch * 16, 16)]
                    for k in range(1, LC):
                        acc = acc + rows_k[r + k, pl.ds(ch * 16, 16)]
                    plsc.store_scatter(out_c, [riota[ch], col],
                                       ctx_m[ch] + acc * inv_lc)

            @pl.when(s + 2 < SPW)
            def _():
                issue(s + 2, gb)

        def loop_body(tt, carry):
            for wb in range(2):
                p = tt * 2 + wb
                s = p * 2

                @pl.when(p >= 2)
                def _():
                    wdesc(p - 2, wb).wait()

                pool_sample(s, 0, wb, 0)
                pool_sample(s + 1, 1, wb, 1)
                wdesc(p, wb).start()
            return carry

        lax.fori_loop(0, SPW // 4, loop_body, 0)

        wdesc(SPW // 2 - 2, 0).wait()
        wdesc(SPW // 2 - 1, 1).wait()

        pltpu.sync_copy(ctxe_v, ctxe_hbm.at[pl.ds(s0, SPW)])

    return sc_kernel(table, ctx_idx, cand_idx)


def _tc_term_head(x, W, b, W2, b2):
    N = x.shape[0]
    H = W.shape[1]
    OD = W2.shape[1]

    def body(x_ref, w_ref, b_ref, w2_ref, b2_ref, o_ref):
        h = jnp.dot(x_ref[...], w_ref[...],
                    preferred_element_type=jnp.float32) + b_ref[...]
        h = jnp.maximum(h, 0.0)
        o_ref[...] = jnp.dot(h, w2_ref[...],
                             preferred_element_type=jnp.float32) + b2_ref[...]

    return pl.pallas_call(
        body,
        grid=(1,),
        in_specs=[
            pl.BlockSpec((N, EMB), lambda i: (0, 0)),
            pl.BlockSpec((EMB, H), lambda i: (0, 0)),
            pl.BlockSpec((1, H), lambda i: (0, 0)),
            pl.BlockSpec((H, OD), lambda i: (0, 0)),
            pl.BlockSpec((1, OD), lambda i: (0, 0)),
        ],
        out_specs=pl.BlockSpec((N, OD), lambda i: (0, 0)),
        out_shape=jax.ShapeDtypeStruct((N, OD), jnp.float32),
    )(x, W, b, W2, b2)


def _tc_cand_head(xT3, Wc1T, bc1c, Wc2T, bc2, block_groups):
    G = xT3.shape[1]
    H = Wc1T.shape[0]
    grid = G // block_groups

    def body(x_ref, w1_ref, b1_ref, w2_ref, b2_ref, o_ref):
        for g in range(block_groups):
            h = jnp.dot(w1_ref[...], x_ref[:, g, :],
                        preferred_element_type=jnp.float32) + b1_ref[...]
            h = jnp.maximum(h, 0.0)
            s = jnp.dot(w2_ref[...], h,
                        preferred_element_type=jnp.float32) + b2_ref[...]
            o_ref[0, g, :] = s[0]

    return pl.pallas_call(
        body,
        grid=(grid,),
        in_specs=[
            pl.BlockSpec((EMB, block_groups, 128), lambda i: (0, i, 0)),
            pl.BlockSpec((H, EMB), lambda i: (0, 0)),
            pl.BlockSpec((H, 1), lambda i: (0, 0)),
            pl.BlockSpec((1, H), lambda i: (0, 0)),
            pl.BlockSpec((1, 1), lambda i: (0, 0)),
        ],
        out_specs=pl.BlockSpec((1, block_groups, 128), lambda i: (i, 0, 0)),
        out_shape=jax.ShapeDtypeStruct((grid, block_groups, 128),
                                       jnp.float32),
    )(xT3, Wc1T, bc1c, Wc2T, bc2)


def kernel(table, W1, b1, W2, b2, Wc1, bc1, Wc2, bc2,
           context_indices, candidate_indices):
    B, LCTX = context_indices.shape
    _, C, LC = candidate_indices.shape

    ctx_i = context_indices.astype(jnp.int32)
    cand_i = candidate_indices.astype(jnp.int32).reshape(B, C * LC)

    combT, ctx_emb = _sc_pool(table, ctx_i, cand_i, B, C, LCTX, LC)
    combT3 = combT.reshape(EMB, B * C // 128, 128)

    term_logits = _tc_term_head(ctx_emb, W1, b1.reshape(1, -1),
                                W2, b2.reshape(1, -1))
    scores = _tc_cand_head(combT3, Wc1.T, bc1.reshape(-1, 1),
                           Wc2.reshape(1, -1), bc2.reshape(1, 1), 64)
    return term_logits, scores.reshape(B, C)


# speedup vs baseline: 1.4781x; 1.0001x over previous
import functools

import jax
import jax.numpy as jnp
from jax import lax
from jax.experimental import pallas as pl
from jax.experimental.pallas import tpu as pltpu
from jax.experimental.pallas import tpu_sc as plsc

EMB = 64
NW = 32
CAND_CHUNKS = ((0, 104), (104, 104), (208, 92))


def _sc_pool(table, ctx_idx, cand_idx, B, C, LCTX, LC):
    SPW = B // NW
    NCI = C * LC
    PC = 2 * C
    PCP = PC
    mesh = plsc.VectorSubcoreMesh(core_axis_name="c", subcore_axis_name="s")

    @functools.partial(
        pl.kernel,
        mesh=mesh,
        compiler_params=pltpu.CompilerParams(use_tc_tiling_on_sc=False,
                                             needs_layout_passes=False),
        out_type=[
            jax.ShapeDtypeStruct((EMB, B * C), jnp.float32),
            jax.ShapeDtypeStruct((B, EMB), jnp.float32),
        ],
        scratch_types=[
            pltpu.VMEM((SPW, LCTX), jnp.int32),
            pltpu.VMEM((SPW, NCI), jnp.int32),
            pltpu.VMEM((2, LCTX, EMB), jnp.float32),
            pltpu.VMEM((2, NCI, EMB), jnp.float32),
            pltpu.VMEM((2, EMB, PCP), jnp.float32),
            pltpu.VMEM((SPW, EMB), jnp.float32),
            pltpu.SemaphoreType.DMA,
            pltpu.SemaphoreType.DMA,
            pltpu.SemaphoreType.DMA,
            pltpu.SemaphoreType.DMA,
        ],
    )
    def sc_kernel(table_hbm, ctx_idx_hbm, cand_idx_hbm,
                  comb_hbm, ctxe_hbm,
                  ctx_idx_v, cand_idx_v, ctx_rows_v, cand_rows_v,
                  comb_v, ctxe_v, gsem0, gsem1, wsem0, wsem1):
        wid = lax.axis_index("s") * 2 + lax.axis_index("c")
        s0 = wid * SPW
        gsems = (gsem0, gsem1)
        wsems = (wsem0, wsem1)

        pltpu.sync_copy(ctx_idx_hbm.at[pl.ds(s0, SPW)], ctx_idx_v)
        pltpu.sync_copy(cand_idx_hbm.at[pl.ds(s0, SPW)], cand_idx_v)

        inv_ctx = jnp.float32(1.0 / LCTX)
        inv_lc = jnp.float32(1.0 / LC)
        zero = jnp.zeros((16,), jnp.float32)
        iota = lax.iota(jnp.int32, 16)
        riota = tuple(iota + 16 * ch for ch in range(4))

        def gather_descs(s, gb):
            descs = [pltpu.make_async_copy(
                table_hbm.at[ctx_idx_v.at[s]], ctx_rows_v.at[gb], gsems[gb])]
            for (off, ln) in CAND_CHUNKS:
                descs.append(pltpu.make_async_copy(
                    table_hbm.at[cand_idx_v.at[s, pl.ds(off, ln)]],
                    cand_rows_v.at[gb, pl.ds(off, ln)], gsems[gb]))
            return descs

        def issue(s, gb):
            for d in gather_descs(s, gb):
                d.start()

        def drain(s, gb):
            for d in gather_descs(s, gb):
                d.wait()

        def wdesc(p, wb):
            return pltpu.make_async_copy(
                comb_v.at[wb, pl.ds(0, EMB), pl.ds(0, PC)],
                comb_hbm.at[pl.ds(0, EMB), pl.ds((s0 + 2 * p) * C, PC)],
                wsems[wb])

        issue(0, 0)
        issue(1, 1)

        def pool_sample(s, gb, wb, u):
            drain(s, gb)

            rows_c = ctx_rows_v.at[gb]
            rows_k = cand_rows_v.at[gb]
            out_c = comb_v.at[wb]

            @plsc.parallel_loop(0, LCTX, 2, unroll=2,
                                carry=(zero,) * 8)
            def sums(i, accs):
                a = tuple(
                    accs[ch] + rows_c[i, pl.ds(ch * 16, 16)]
                    for ch in range(4))
                b = tuple(
                    accs[4 + ch] + rows_c[i + 1, pl.ds(ch * 16, 16)]
                    for ch in range(4))
                return a + b
            ctx_m = tuple(
                (sums[ch] + sums[4 + ch]) * inv_ctx for ch in range(4))
            for ch in range(4):
                ctxe_v[s, pl.ds(ch * 16, 16)] = ctx_m[ch]

            @plsc.parallel_loop(0, C, 1, unroll=4)
            def _(cidx):
                r = cidx * LC
                col = jnp.full((16,), u * C + cidx, jnp.int32)
                for ch in range(4):
                    acc = rows_k[r, pl.ds(ch * 16, 16)]
                    for k in range(1, LC):
                        acc = acc + rows_k[r + k, pl.ds(ch * 16, 16)]
                    plsc.store_scatter(out_c, [riota[ch], col],
                                       ctx_m[ch] + acc * inv_lc)

            @pl.when(s + 2 < SPW)
            def _():
                issue(s + 2, gb)

        def loop_body(tt, carry):
            for wb in range(2):
                p = tt * 2 + wb
                s = p * 2

                @pl.when(p >= 2)
                def _():
                    wdesc(p - 2, wb).wait()

                pool_sample(s, 0, wb, 0)
                pool_sample(s + 1, 1, wb, 1)
                wdesc(p, wb).start()
            return carry

        lax.fori_loop(0, SPW // 4, loop_body, 0)

        wdesc(SPW // 2 - 2, 0).wait()
        wdesc(SPW // 2 - 1, 1).wait()

        pltpu.sync_copy(ctxe_v, ctxe_hbm.at[pl.ds(s0, SPW)])

    return sc_kernel(table, ctx_idx, cand_idx)


def _tc_term_head(x, W, b, W2, b2):
    N = x.shape[0]
    H = W.shape[1]
    OD = W2.shape[1]

    def body(x_ref, w_ref, b_ref, w2_ref, b2_ref, o_ref):
        h = jnp.dot(x_ref[...], w_ref[...],
                    preferred_element_type=jnp.float32) + b_ref[...]
        h = jnp.maximum(h, 0.0)
        o_ref[...] = jnp.dot(h, w2_ref[...],
                             preferred_element_type=jnp.float32) + b2_ref[...]

    return pl.pallas_call(
        body,
        grid=(1,),
        in_specs=[
            pl.BlockSpec((N, EMB), lambda i: (0, 0)),
            pl.BlockSpec((EMB, H), lambda i: (0, 0)),
            pl.BlockSpec((1, H), lambda i: (0, 0)),
            pl.BlockSpec((H, OD), lambda i: (0, 0)),
            pl.BlockSpec((1, OD), lambda i: (0, 0)),
        ],
        out_specs=pl.BlockSpec((N, OD), lambda i: (0, 0)),
        out_shape=jax.ShapeDtypeStruct((N, OD), jnp.float32),
    )(x, W, b, W2, b2)


def _tc_cand_head(xT3, Wc1T, bc1c, Wc2T, bc2, block_groups):
    G = xT3.shape[1]
    H = Wc1T.shape[0]
    grid = G // block_groups

    def body(x_ref, w1_ref, b1_ref, w2_ref, b2_ref, o_ref):
        for g in range(block_groups):
            h = jnp.dot(w1_ref[...], x_ref[:, g, :],
                        preferred_element_type=jnp.float32) + b1_ref[...]
            h = jnp.maximum(h, 0.0)
            s = jnp.dot(w2_ref[...], h,
                        preferred_element_type=jnp.float32) + b2_ref[...]
            o_ref[0, g, :] = s[0]

    return pl.pallas_call(
        body,
        grid=(grid,),
        in_specs=[
            pl.BlockSpec((EMB, block_groups, 128), lambda i: (0, i, 0)),
            pl.BlockSpec((H, EMB), lambda i: (0, 0)),
            pl.BlockSpec((H, 1), lambda i: (0, 0)),
            pl.BlockSpec((1, H), lambda i: (0, 0)),
            pl.BlockSpec((1, 1), lambda i: (0, 0)),
        ],
        out_specs=pl.BlockSpec((1, block_groups, 128), lambda i: (i, 0, 0)),
        out_shape=jax.ShapeDtypeStruct((grid, block_groups, 128),
                                       jnp.float32),
    )(xT3, Wc1T, bc1c, Wc2T, bc2)


def kernel(table, W1, b1, W2, b2, Wc1, bc1, Wc2, bc2,
           context_indices, candidate_indices):
    B, LCTX = context_indices.shape
    _, C, LC = candidate_indices.shape

    ctx_i = context_indices.astype(jnp.int32)
    cand_i = candidate_indices.astype(jnp.int32).reshape(B, C * LC)

    combT, ctx_emb = _sc_pool(table, ctx_i, cand_i, B, C, LCTX, LC)
    combT3 = combT.reshape(EMB, B * C // 128, 128)

    term_logits = _tc_term_head(ctx_emb, W1, b1.reshape(1, -1),
                                W2, b2.reshape(1, -1))
    scores = _tc_cand_head(combT3, Wc1.T, bc1.reshape(-1, 1),
                           Wc2.reshape(1, -1), bc2.reshape(1, 1), 128)
    return term_logits, scores.reshape(B, C)
